# speedup vs baseline: 2.9692x; 2.9692x over previous
import functools

import jax
import jax.numpy as jnp
from jax import lax
from jax.experimental import pallas as pl
from jax.experimental.pallas import tpu as pltpu
from jax.experimental.pallas import tpu_sc as plsc

_VOCAB = 100000
_D = 128
_B = 4096
_H = 50
_TOTAL = _B * _H
_NW = 32
_B_PER_W = _TOTAL // _NW
_CHUNK = 128
_NCHUNK = _B_PER_W // _CHUNK


def _emb_body(idx_hbm, table_hbm, out_hbm, idx_v, rows_v, sem):
    info = plsc.get_sparse_core_info()
    nc = info.num_cores
    wid = lax.axis_index("s") * nc + lax.axis_index("c")
    base = wid * _B_PER_W
    pltpu.sync_copy(idx_hbm.at[wid], idx_v)

    def body(c, carry):
        pltpu.async_copy(table_hbm.at[idx_v.at[c]], rows_v, sem).wait()
        pltpu.sync_copy(rows_v, out_hbm.at[pl.ds(base + c * _CHUNK, _CHUNK)])
        return carry

    lax.fori_loop(0, _NCHUNK, body, 0)


@jax.jit
def _emb(idx, table):
    k = functools.partial(
        pl.kernel,
        mesh=plsc.VectorSubcoreMesh(core_axis_name="c", subcore_axis_name="s"),
        out_type=jax.ShapeDtypeStruct((_TOTAL, _D), jnp.float32),
        scratch_types=[
            pltpu.VMEM((_NCHUNK, _CHUNK), jnp.int32),
            pltpu.VMEM((_CHUNK, _D), jnp.float32),
            pltpu.SemaphoreType.DMA,
        ],
    )(_emb_body)
    return k(idx, table)


def kernel(input_tensor, table):
    idx = input_tensor.reshape(_NW, _NCHUNK, _CHUNK).astype(jnp.int32)
    out = _emb(idx, table)
    return out.reshape(_B, _H, _D)


# speedup vs baseline: 3.3079x; 1.1141x over previous
import functools

import jax
import jax.numpy as jnp
from jax import lax
from jax.experimental import pallas as pl
from jax.experimental.pallas import tpu as pltpu
from jax.experimental.pallas import tpu_sc as plsc

_VOCAB = 100000
_D = 128
_B = 4096
_H = 50
_TOTAL = _B * _H
_NW = 32
_B_PER_W = _TOTAL // _NW
_CHUNK = 128
_NCHUNK = _B_PER_W // _CHUNK
_NBUF = 5
_ROUNDS = _NCHUNK // _NBUF


def _emb_body(idx_hbm, table_hbm, out_hbm, idx_v, *scr):
    bufs = scr[:_NBUF]
    gsems = scr[_NBUF:2 * _NBUF]
    wsems = scr[2 * _NBUF:3 * _NBUF]
    info = plsc.get_sparse_core_info()
    wid = lax.axis_index("s") * info.num_cores + lax.axis_index("c")
    base = wid * _B_PER_W
    pltpu.sync_copy(idx_hbm.at[wid], idx_v)

    def gather(c, b):
        pltpu.async_copy(table_hbm.at[idx_v.at[c]], bufs[b], gsems[b])

    def gather_wait(c, b):
        pltpu.make_async_copy(table_hbm.at[idx_v.at[c]], bufs[b],
                              gsems[b]).wait()

    def write(c, b):
        pltpu.async_copy(bufs[b],
                         out_hbm.at[pl.ds(base + c * _CHUNK, _CHUNK)],
                         wsems[b])

    def write_wait(c, b):
        pltpu.make_async_copy(bufs[b],
                              out_hbm.at[pl.ds(base + c * _CHUNK, _CHUNK)],
                              wsems[b]).wait()

    for b in range(_NBUF):
        gather(b, b)

    def round_body(r, carry):
        c0 = r * _NBUF
        for b in range(_NBUF):
            gather_wait(c0 + b, b)
            write(c0 + b, b)
        for b in range(_NBUF):
            write_wait(c0 + b, b)
            gather(c0 + _NBUF + b, b)
        return carry

    lax.fori_loop(0, _ROUNDS - 1, round_body, 0)

    c0 = (_ROUNDS - 1) * _NBUF
    for b in range(_NBUF):
        gather_wait(c0 + b, b)
        write(c0 + b, b)
    for b in range(_NBUF):
        write_wait(c0 + b, b)


@jax.jit
def _emb(idx, table):
    k = functools.partial(
        pl.kernel,
        mesh=plsc.VectorSubcoreMesh(core_axis_name="c", subcore_axis_name="s"),
        out_type=jax.ShapeDtypeStruct((_TOTAL, _D), jnp.float32),
        scratch_types=(
            [pltpu.VMEM((_NCHUNK, _CHUNK), jnp.int32)]
            + [pltpu.VMEM((_CHUNK, _D), jnp.float32) for _ in range(_NBUF)]
            + [pltpu.SemaphoreType.DMA for _ in range(2 * _NBUF)]
        ),
    )(_emb_body)
    return k(idx, table)


def kernel(input_tensor, table):
    idx = input_tensor.reshape(_NW, _NCHUNK, _CHUNK).astype(jnp.int32)
    out = _emb(idx, table)
    return out.reshape(_B, _H, _D)


# speedup vs baseline: 5.8857x; 1.7793x over previous
import functools

import jax
import jax.numpy as jnp
from jax import lax
from jax.experimental import pallas as pl
from jax.experimental.pallas import tpu as pltpu
from jax.experimental.pallas import tpu_sc as plsc

_VOCAB = 100000
_D = 128
_B = 4096
_H = 50
_NW = 32
_ROWS_PER_W = _B // _NW
_RPC = 2
_CHUNK = _RPC * _H
_NCHUNK = _ROWS_PER_W // _RPC
_NBUF = 4
_ROUNDS = _NCHUNK // _NBUF


def _emb_body(idx_hbm, table_hbm, out_hbm, idx_v, *scr):
    bufs = scr[:_NBUF]
    gsems = scr[_NBUF:2 * _NBUF]
    wsems = scr[2 * _NBUF:3 * _NBUF]
    info = plsc.get_sparse_core_info()
    wid = lax.axis_index("s") * info.num_cores + lax.axis_index("c")
    row0 = wid * _ROWS_PER_W
    pltpu.sync_copy(idx_hbm.at[wid], idx_v)

    def gather(c, b):
        pltpu.async_copy(table_hbm.at[idx_v.at[c]], bufs[b], gsems[b])

    def gather_wait(c, b):
        pltpu.make_async_copy(table_hbm.at[idx_v.at[c]], bufs[b],
                              gsems[b]).wait()

    def write(c, b):
        for r in range(_RPC):
            pltpu.async_copy(bufs[b].at[pl.ds(r * _H, _H)],
                             out_hbm.at[row0 + c * _RPC + r],
                             wsems[b])

    def write_wait(c, b):
        for r in range(_RPC):
            pltpu.make_async_copy(bufs[b].at[pl.ds(r * _H, _H)],
                                  out_hbm.at[row0 + c * _RPC + r],
                                  wsems[b]).wait()

    for b in range(_NBUF):
        gather(b, b)

    def round_body(r, carry):
        c0 = r * _NBUF
        for b in range(_NBUF):
            gather_wait(c0 + b, b)
            write(c0 + b, b)
        for b in range(_NBUF):
            write_wait(c0 + b, b)
            gather(c0 + _NBUF + b, b)
        return carry

    lax.fori_loop(0, _ROUNDS - 1, round_body, 0)

    c0 = (_ROUNDS - 1) * _NBUF
    for b in range(_NBUF):
        gather_wait(c0 + b, b)
        write(c0 + b, b)
    for b in range(_NBUF):
        write_wait(c0 + b, b)


@jax.jit
def _emb(idx, table):
    k = functools.partial(
        pl.kernel,
        mesh=plsc.VectorSubcoreMesh(core_axis_name="c", subcore_axis_name="s"),
        out_type=jax.ShapeDtypeStruct((_B, _H, _D), jnp.float32),
        scratch_types=(
            [pltpu.VMEM((_NCHUNK, _CHUNK), jnp.int32)]
            + [pltpu.VMEM((_CHUNK, _D), jnp.float32) for _ in range(_NBUF)]
            + [pltpu.SemaphoreType.DMA for _ in range(2 * _NBUF)]
        ),
    )(_emb_body)
    return k(idx, table)


def kernel(input_tensor, table):
    idx = input_tensor.reshape(_NW, _NCHUNK, _CHUNK).astype(jnp.int32)
    return _emb(idx, table)


# speedup vs baseline: 10.1183x; 1.7191x over previous
import functools

import jax
import jax.numpy as jnp
from jax import lax
from jax.experimental import pallas as pl
from jax.experimental.pallas import tpu as pltpu
from jax.experimental.pallas import tpu_sc as plsc

_VOCAB = 100000
_D = 128
_B = 4096
_H = 50
_TOTAL = _B * _H
_NW = 32
_B_PER_W = _TOTAL // _NW
_CHUNK = 128
_NCHUNK = _B_PER_W // _CHUNK
_NBUF = 5
_ROUNDS = _NCHUNK // _NBUF


def _emb_body(idx_hbm, table_hbm, out_hbm, idx_v, *scr):
    bufs = scr[:_NBUF]
    gsems = scr[_NBUF:2 * _NBUF]
    wsems = scr[2 * _NBUF:3 * _NBUF]
    info = plsc.get_sparse_core_info()
    wid = lax.axis_index("s") * info.num_cores + lax.axis_index("c")
    base = wid * _B_PER_W
    pltpu.sync_copy(idx_hbm.at[wid], idx_v)

    def out_slice(c):
        j0 = base + c * _CHUNK
        return out_hbm.at[j0 // _B, pl.ds(j0 % _B, _CHUNK)]

    def gather(c, b):
        pltpu.async_copy(table_hbm.at[idx_v.at[c]], bufs[b], gsems[b])

    def gather_wait(c, b):
        pltpu.make_async_copy(table_hbm.at[idx_v.at[c]], bufs[b],
                              gsems[b]).wait()

    def write(c, b):
        pltpu.async_copy(bufs[b], out_slice(c), wsems[b])

    def write_wait(c, b):
        pltpu.make_async_copy(bufs[b], out_slice(c), wsems[b]).wait()

    for b in range(_NBUF):
        gather(b, b)

    def round_body(r, carry):
        c0 = r * _NBUF
        for b in range(_NBUF):
            gather_wait(c0 + b, b)
            write(c0 + b, b)
        for b in range(_NBUF):
            write_wait(c0 + b, b)
            gather(c0 + _NBUF + b, b)
        return carry

    lax.fori_loop(0, _ROUNDS - 1, round_body, 0)

    c0 = (_ROUNDS - 1) * _NBUF
    for b in range(_NBUF):
        gather_wait(c0 + b, b)
        write(c0 + b, b)
    for b in range(_NBUF):
        write_wait(c0 + b, b)


@jax.jit
def _emb(idx, table):
    k = functools.partial(
        pl.kernel,
        mesh=plsc.VectorSubcoreMesh(core_axis_name="c", subcore_axis_name="s"),
        out_type=jax.ShapeDtypeStruct((_H, _B, _D), jnp.float32),
        scratch_types=(
            [pltpu.VMEM((_NCHUNK, _CHUNK), jnp.int32)]
            + [pltpu.VMEM((_CHUNK, _D), jnp.float32) for _ in range(_NBUF)]
            + [pltpu.SemaphoreType.DMA for _ in range(2 * _NBUF)]
        ),
    )(_emb_body)
    return k(idx, table)


def kernel(input_tensor, table):
    idx = input_tensor.T.reshape(_NW, _NCHUNK, _CHUNK).astype(jnp.int32)
    out_t = _emb(idx, table)
    return jnp.transpose(out_t, (1, 0, 2))


# speedup vs baseline: 10.4482x; 1.0326x over previous
import functools

import jax
import jax.numpy as jnp
from jax import lax
from jax.experimental import pallas as pl
from jax.experimental.pallas import tpu as pltpu
from jax.experimental.pallas import tpu_sc as plsc

_VOCAB = 100000
_D = 128
_B = 4096
_H = 50
_TOTAL = _B * _H
_NW = 32
_B_PER_W = _TOTAL // _NW
_CHUNK = 128
_NCHUNK = _B_PER_W // _CHUNK
_NBUF = 5
_LOOK = 2
_ROUNDS = _NCHUNK // _NBUF


def _emb_body(idx_hbm, table_hbm, out_hbm, idx_v, *scr):
    bufs = scr[:_NBUF]
    gsems = scr[_NBUF:2 * _NBUF]
    wsems = scr[2 * _NBUF:3 * _NBUF]
    info = plsc.get_sparse_core_info()
    wid = lax.axis_index("s") * info.num_cores + lax.axis_index("c")
    base = wid * _B_PER_W
    pltpu.sync_copy(idx_hbm.at[wid], idx_v)

    def out_slice(c):
        j0 = base + c * _CHUNK
        return out_hbm.at[j0 // _B, pl.ds(j0 % _B, _CHUNK)]

    def gather(c, b):
        pltpu.async_copy(table_hbm.at[idx_v.at[c]], bufs[b], gsems[b])

    def gather_wait(c, b):
        pltpu.make_async_copy(table_hbm.at[idx_v.at[c]], bufs[b],
                              gsems[b]).wait()

    def write(c, b):
        pltpu.async_copy(bufs[b], out_slice(c), wsems[b])

    def write_wait(c, b):
        pltpu.make_async_copy(bufs[b], out_slice(c), wsems[b]).wait()

    for b in range(_LOOK):
        gather(b, b)

    for b in range(_NBUF):
        s = (b + _LOOK) % _NBUF
        if b >= _NBUF - _LOOK:
            write_wait(b - (_NBUF - _LOOK), s)
        gather(b + _LOOK, s)
        gather_wait(b, b)
        write(b, b)

    def round_body(r, carry):
        c0 = r * _NBUF
        for b in range(_NBUF):
            c = c0 + b
            s = (b + _LOOK) % _NBUF
            write_wait(c - (_NBUF - _LOOK), s)
            gather(c + _LOOK, s)
            gather_wait(c, b)
            write(c, b)
        return carry

    lax.fori_loop(1, _ROUNDS - 1, round_body, 0)

    c0 = (_ROUNDS - 1) * _NBUF
    for b in range(_NBUF):
        c = c0 + b
        s = (b + _LOOK) % _NBUF
        write_wait(c - (_NBUF - _LOOK), s)
        if b < _NBUF - _LOOK:
            gather(c + _LOOK, s)
        gather_wait(c, b)
        write(c, b)

    for b in range(_LOOK, _NBUF):
        write_wait(c0 + b, b)


@jax.jit
def _emb(idx, table):
    k = functools.partial(
        pl.kernel,
        mesh=plsc.VectorSubcoreMesh(core_axis_name="c", subcore_axis_name="s"),
        out_type=jax.ShapeDtypeStruct((_H, _B, _D), jnp.float32),
        scratch_types=(
            [pltpu.VMEM((_NCHUNK, _CHUNK), jnp.int32)]
            + [pltpu.VMEM((_CHUNK, _D), jnp.float32) for _ in range(_NBUF)]
            + [pltpu.SemaphoreType.DMA for _ in range(2 * _NBUF)]
        ),
    )(_emb_body)
    return k(idx, table)


def kernel(input_tensor, table):
    idx = input_tensor.T.reshape(_NW, _NCHUNK, _CHUNK).astype(jnp.int32)
    out_t = _emb(idx, table)
    return jnp.transpose(out_t, (1, 0, 2))
